# baseline (speedup 1.0000x reference)
import functools
import math

import jax
import jax.numpy as jnp
from jax import lax
from jax.experimental import pallas as pl

_B = 16384
_STATUS = 128
_QL = 50
_EF = 16
_H = 4
_DH = 32
_GF = 1600
_HID = 64
_OUT = 64

_HI = lax.Precision.HIGHEST


def _gumbel_noise(bsz):
    u = jax.random.uniform(jax.random.key(42), (bsz, _QL), dtype=jnp.float32)
    return -jnp.log(-jnp.log(u + 1e-20) + 1e-20)


def _body(vs_ref, img_ref, g_ref, wq_ref, wkt_ref, w1_ref, b1_ref, w2_ref,
          b2_ref, out_ref):
    vs = vs_ref[...]
    status = vs[:, :_STATUS]
    em = vs[:, _STATUS:]

    qf = jnp.dot(status, wq_ref[...], precision=_HI)
    t = jnp.dot(qf, wkt_ref[...], precision=_HI) / jnp.float32(
        _H * math.sqrt(_DH))

    col_f = lax.broadcasted_iota(jnp.int32, (_EF, _QL * _EF), 1)
    row_f = lax.broadcasted_iota(jnp.int32, (_EF, _QL * _EF), 0)
    rep_t = (lax.rem(col_f, _EF) == row_f).astype(jnp.float32)

    col_q = lax.broadcasted_iota(jnp.int32, (_QL * _EF, _QL), 0)
    q_q = lax.broadcasted_iota(jnp.int32, (_QL * _EF, _QL), 1)
    seg = (col_q // _EF == q_q).astype(jnp.float32)

    t_rep = jnp.dot(t, rep_t, precision=_HI)
    wm = jnp.dot(em * t_rep, seg, precision=_HI)

    nz = (em != 0.0).astype(jnp.float32)
    cnt = jnp.dot(nz, seg, precision=_HI)
    wm = jnp.where(cnt == 0.0, jnp.float32(-1e8), wm)

    logits = (wm + jnp.float32(1e-8) + g_ref[...]) / jnp.float32(0.1)
    m = jnp.max(logits, axis=-1, keepdims=True)
    e = jnp.exp(logits - m)
    sel = e / jnp.sum(e, axis=-1, keepdims=True)

    sel_rep = jnp.dot(sel, seg.T, precision=_HI)
    selected = jnp.dot(em * sel_rep, rep_t.T, precision=_HI)

    w1 = w1_ref[...]
    h = (jnp.dot(status, w1[:_STATUS], precision=_HI)
         + jnp.dot(selected, w1[_STATUS:_STATUS + _EF], precision=_HI)
         + jnp.dot(img_ref[...], w1[_STATUS + _EF:], precision=_HI)
         + b1_ref[...])
    h = jnp.maximum(h, 0.0)
    out = jnp.maximum(jnp.dot(h, w2_ref[...], precision=_HI) + b2_ref[...], 0.0)
    out_ref[...] = out


@jax.jit
def kernel(vector_state, image_state, Wq, Wk, Wv, W1, b1, W2, b2):
    del Wv
    bsz = vector_state.shape[0]
    img = image_state.reshape(bsz, _GF)
    g = _gumbel_noise(bsz)
    wkt = Wk.T
    b1r = b1.reshape(1, _HID)
    b2r = b2.reshape(1, _OUT)

    bt = 512
    grid = (bsz // bt,)
    row = lambda i: (i, 0)
    rep = lambda i: (0, 0)
    return pl.pallas_call(
        _body,
        grid=grid,
        in_specs=[
            pl.BlockSpec((bt, _STATUS + _QL * _EF), row),
            pl.BlockSpec((bt, _GF), row),
            pl.BlockSpec((bt, _QL), row),
            pl.BlockSpec(Wq.shape, rep),
            pl.BlockSpec(wkt.shape, rep),
            pl.BlockSpec(W1.shape, rep),
            pl.BlockSpec(b1r.shape, rep),
            pl.BlockSpec(W2.shape, rep),
            pl.BlockSpec(b2r.shape, rep),
        ],
        out_specs=pl.BlockSpec((bt, _OUT), row),
        out_shape=jax.ShapeDtypeStruct((bsz, _OUT), jnp.float32),
    )(vector_state, img, g, Wq, wkt, W1, b1r, W2, b2r)


# speedup vs baseline: 2.3729x; 2.3729x over previous
import functools
import math

import jax
import jax.numpy as jnp
from jax import lax
from jax.experimental import pallas as pl

_B = 16384
_STATUS = 128
_QL = 50
_EF = 16
_H = 4
_DH = 32
_GF = 1600
_HID = 64
_OUT = 64

_HI = lax.Precision.HIGHEST
_DP = lax.Precision.DEFAULT


def _gumbel_noise(bsz):
    u = jax.random.uniform(jax.random.key(42), (bsz, _QL), dtype=jnp.float32)
    return -jnp.log(-jnp.log(u + 1e-20) + 1e-20)


def _body(vs_ref, img_ref, g_ref, wq_ref, wkt_ref, w1_ref, b1_ref, w2_ref,
          b2_ref, out_ref):
    vs = vs_ref[...]
    status = vs[:, :_STATUS]
    em = vs[:, _STATUS:]

    qf = jnp.dot(status, wq_ref[...], precision=_DP)
    t = jnp.dot(qf, wkt_ref[...], precision=_DP) / jnp.float32(
        _H * math.sqrt(_DH))

    col_f = lax.broadcasted_iota(jnp.int32, (_EF, _QL * _EF), 1)
    row_f = lax.broadcasted_iota(jnp.int32, (_EF, _QL * _EF), 0)
    rep_t = (lax.rem(col_f, _EF) == row_f).astype(jnp.float32)

    col_q = lax.broadcasted_iota(jnp.int32, (_QL * _EF, _QL), 0)
    q_q = lax.broadcasted_iota(jnp.int32, (_QL * _EF, _QL), 1)
    seg = (col_q // _EF == q_q).astype(jnp.float32)

    t_rep = jnp.dot(t, rep_t, precision=_DP)
    wm = jnp.dot(em * t_rep, seg, precision=_DP)

    nz = (em != 0.0).astype(jnp.float32)
    cnt = jnp.dot(nz, seg, precision=_DP)
    wm = jnp.where(cnt == 0.0, jnp.float32(-1e8), wm)

    logits = (wm + jnp.float32(1e-8) + g_ref[...]) / jnp.float32(0.1)
    m = jnp.max(logits, axis=-1, keepdims=True)
    e = jnp.exp(logits - m)
    sel = e / jnp.sum(e, axis=-1, keepdims=True)

    sel_rep = jnp.dot(sel, seg.T, precision=_DP)
    selected = jnp.dot(em * sel_rep, rep_t.T, precision=_DP)

    w1 = w1_ref[...]
    h = (jnp.dot(status, w1[:_STATUS], precision=_DP)
         + jnp.dot(selected, w1[_STATUS:_STATUS + _EF], precision=_DP)
         + jnp.dot(img_ref[...], w1[_STATUS + _EF:], precision=_DP)
         + b1_ref[...])
    h = jnp.maximum(h, 0.0)
    out = jnp.maximum(jnp.dot(h, w2_ref[...], precision=_DP) + b2_ref[...], 0.0)
    out_ref[...] = out


@jax.jit
def kernel(vector_state, image_state, Wq, Wk, Wv, W1, b1, W2, b2):
    del Wv
    bsz = vector_state.shape[0]
    img = image_state.reshape(bsz, _GF)
    g = _gumbel_noise(bsz)
    wkt = Wk.T
    b1r = b1.reshape(1, _HID)
    b2r = b2.reshape(1, _OUT)

    bt = 512
    grid = (bsz // bt,)
    row = lambda i: (i, 0)
    rep = lambda i: (0, 0)
    return pl.pallas_call(
        _body,
        grid=grid,
        in_specs=[
            pl.BlockSpec((bt, _STATUS + _QL * _EF), row),
            pl.BlockSpec((bt, _GF), row),
            pl.BlockSpec((bt, _QL), row),
            pl.BlockSpec(Wq.shape, rep),
            pl.BlockSpec(wkt.shape, rep),
            pl.BlockSpec(W1.shape, rep),
            pl.BlockSpec(b1r.shape, rep),
            pl.BlockSpec(W2.shape, rep),
            pl.BlockSpec(b2r.shape, rep),
        ],
        out_specs=pl.BlockSpec((bt, _OUT), row),
        out_shape=jax.ShapeDtypeStruct((bsz, _OUT), jnp.float32),
    )(vector_state, img, g, Wq, wkt, W1, b1r, W2, b2r)
